# baseline (speedup 1.0000x reference)
import functools

import jax
import jax.numpy as jnp
from jax import lax
from jax.experimental import pallas as pl
from jax.experimental.pallas import tpu as pltpu
from jax.experimental.pallas import tpu_sc as plsc

D = 768
F = 2048
E = 8
T = 2048
TM = 128
NT = 40
P = NT * TM
FB = 512
NC = 2
NS = 16
NW = NC * NS
TPW = T // NW
S = 32
NEG = -1e30


def _router_body(wr_ref, x_ref, bias_ref, w1_ref, w2_ref, p0_ref, p1_ref,
                 te_ref):
    lT = lax.dot_general(wr_ref[...], x_ref[...], (((1,), (1,)), ((), ())),
                         preferred_element_type=jnp.float32)
    lT = lT + bias_ref[...]
    er = lax.broadcasted_iota(jnp.int32, lT.shape, 0)
    m1 = jnp.max(lT, axis=0, keepdims=True)
    a1 = jnp.min(jnp.where(lT == m1, er, E), axis=0, keepdims=True)
    oh1 = er == a1
    l2 = jnp.where(oh1, NEG, lT)
    m2 = jnp.max(l2, axis=0, keepdims=True)
    a2 = jnp.min(jnp.where(l2 == m2, er, E), axis=0, keepdims=True)
    oh2 = er == a2
    e21 = jnp.exp(m2 - m1)
    w1 = 1.0 / (1.0 + e21)
    w1_ref[...] = w1
    w2_ref[...] = 1.0 - w1

    C = jnp.where(jnp.logical_or(oh1, oh2), 1.0, 0.0)
    r = lax.broadcasted_iota(jnp.int32, (T, T), 0)
    c = lax.broadcasted_iota(jnp.int32, (T, T), 1)
    U = jnp.where(r < c, 1.0, 0.0)
    Cex = lax.dot_general(C, U, (((1,), (0,)), ((), ())),
                          preferred_element_type=jnp.float32)
    tot = jnp.max(Cex + C, axis=1, keepdims=True)
    pc = jnp.ceil(tot * (1.0 / TM)) * TM
    e8r = lax.broadcasted_iota(jnp.int32, (E, E), 0)
    e8c = lax.broadcasted_iota(jnp.int32, (E, E), 1)
    L8 = jnp.where(e8c < e8r, 1.0, 0.0)
    start = lax.dot_general(L8, pc, (((1,), (0,)), ((), ())),
                            preferred_element_type=jnp.float32)
    posbase = start + Cex
    p0 = jnp.sum(jnp.where(oh1, posbase, 0.0), axis=0, keepdims=True)
    p1 = jnp.sum(jnp.where(oh2, posbase, 0.0), axis=0, keepdims=True)
    p0_ref[...] = p0.astype(jnp.int32)
    p1_ref[...] = p1.astype(jnp.int32)

    ti = lax.broadcasted_iota(jnp.int32, (E, 128), 1).astype(jnp.float32) * TM
    inm = jnp.logical_and(ti >= start, ti < start + pc)
    ev = lax.broadcasted_iota(jnp.int32, (E, 128), 0).astype(jnp.float32)
    te = 7.0 + jnp.sum(jnp.where(inm, ev - 7.0, 0.0), axis=0, keepdims=True)
    te_ref[...] = te.astype(jnp.int32)


def _run_router(xf, W_router, expert_bias):
    return pl.pallas_call(
        _router_body,
        out_shape=[
            jax.ShapeDtypeStruct((1, T), jnp.float32),
            jax.ShapeDtypeStruct((1, T), jnp.float32),
            jax.ShapeDtypeStruct((1, T), jnp.int32),
            jax.ShapeDtypeStruct((1, T), jnp.int32),
            jax.ShapeDtypeStruct((1, 128), jnp.int32),
        ],
    )(W_router, xf, expert_bias.reshape(E, 1))


def _sc_scatter(xf, p0, p1, w0, w1):
    mesh = plsc.VectorSubcoreMesh(core_axis_name="c", subcore_axis_name="s")

    @functools.partial(
        pl.kernel,
        mesh=mesh,
        out_type=[
            jax.ShapeDtypeStruct((P, D), jnp.float32),
            jax.ShapeDtypeStruct((P,), jnp.float32),
        ],
        scratch_types=[
            pltpu.VMEM((TPW, D), jnp.float32),
            pltpu.VMEM((TPW,), jnp.int32),
            pltpu.VMEM((TPW,), jnp.int32),
            pltpu.VMEM((TPW,), jnp.float32),
            pltpu.VMEM((TPW,), jnp.float32),
            pltpu.SemaphoreType.DMA,
        ],
    )
    def k(x_hbm, p0_hbm, p1_hbm, w0_hbm, w1_hbm, xs_hbm, rw_hbm,
          rows_v, i0_v, i1_v, w0_v, w1_v, sem):
        wid = lax.axis_index("s") * NC + lax.axis_index("c")
        base = wid * TPW
        pltpu.sync_copy(x_hbm.at[pl.ds(base, TPW)], rows_v)
        pltpu.sync_copy(p0_hbm.at[pl.ds(base, TPW)], i0_v)
        pltpu.sync_copy(p1_hbm.at[pl.ds(base, TPW)], i1_v)
        pltpu.sync_copy(w0_hbm.at[pl.ds(base, TPW)], w0_v)
        pltpu.sync_copy(w1_hbm.at[pl.ds(base, TPW)], w1_v)
        c0 = pltpu.async_copy(rows_v, xs_hbm.at[i0_v], sem)
        c0.wait()
        c1 = pltpu.async_copy(rows_v, xs_hbm.at[i1_v], sem)
        c1.wait()
        c2 = pltpu.async_copy(w0_v, rw_hbm.at[i0_v], sem)
        c2.wait()
        c3 = pltpu.async_copy(w1_v, rw_hbm.at[i1_v], sem)
        c3.wait()

    return k(xf, p0, p1, w0, w1)


def _ffn_body(te_ref, xs_ref, rw_ref, wg_ref, wu_ref, wd_ref, y_ref):
    del te_ref
    xt = xs_ref[...]
    g = lax.dot_general(xt, wg_ref[0], (((1,), (1,)), ((), ())),
                        preferred_element_type=jnp.float32)
    u = lax.dot_general(xt, wu_ref[0], (((1,), (1,)), ((), ())),
                        preferred_element_type=jnp.float32)
    h = g * jax.nn.sigmoid(g) * u
    y = lax.dot_general(h, wd_ref[0], (((1,), (1,)), ((), ())),
                        preferred_element_type=jnp.float32)
    rr = lax.broadcasted_iota(jnp.int32, (TM, TM), 0)
    cc = lax.broadcasted_iota(jnp.int32, (TM, TM), 1)
    dw = jnp.where(rr == cc, jnp.broadcast_to(rw_ref[0], (TM, TM)), 0.0)
    y_ref[...] = lax.dot_general(dw, y, (((1,), (0,)), ((), ())),
                                 preferred_element_type=jnp.float32)


def _run_ffn(te, xs, rw, Wg, Wu, Wd):
    return pl.pallas_call(
        _ffn_body,
        grid_spec=pltpu.PrefetchScalarGridSpec(
            num_scalar_prefetch=1,
            grid=(NT,),
            in_specs=[
                pl.BlockSpec((TM, D), lambda i, te: (i, 0)),
                pl.BlockSpec((1, 1, TM), lambda i, te: (i, 0, 0)),
                pl.BlockSpec((1, F, D), lambda i, te: (te[i], 0, 0)),
                pl.BlockSpec((1, F, D), lambda i, te: (te[i], 0, 0)),
                pl.BlockSpec((1, D, F), lambda i, te: (te[i], 0, 0)),
            ],
            out_specs=pl.BlockSpec((TM, D), lambda i, te: (i, 0)),
        ),
        out_shape=jax.ShapeDtypeStruct((P, D), jnp.float32),
    )(te, xs, rw.reshape(NT, 1, TM), Wg, Wu, Wd)


def _shared_body(x_ref, wg_ref, wu_ref, wd_ref, out_ref):
    f = pl.program_id(0)
    xt = x_ref[...]
    g = lax.dot_general(xt, wg_ref[0], (((1,), (1,)), ((), ())),
                        preferred_element_type=jnp.float32)
    u = lax.dot_general(xt, wu_ref[0], (((1,), (1,)), ((), ())),
                        preferred_element_type=jnp.float32)
    h = g * jax.nn.sigmoid(g) * u
    contrib = lax.dot_general(h, wd_ref[0], (((1,), (1,)), ((), ())),
                              preferred_element_type=jnp.float32)

    @pl.when(f == 0)
    def _():
        out_ref[...] = jnp.zeros_like(out_ref)

    out_ref[...] += contrib


def _run_shared(xf, shared_Wg, shared_Wu, shared_Wd):
    nf = F // FB
    return pl.pallas_call(
        _shared_body,
        grid=(nf,),
        in_specs=[
            pl.BlockSpec((T, D), lambda f: (0, 0)),
            pl.BlockSpec((1, FB, D), lambda f: (0, f, 0)),
            pl.BlockSpec((1, FB, D), lambda f: (0, f, 0)),
            pl.BlockSpec((1, D, FB), lambda f: (0, 0, f)),
        ],
        out_specs=pl.BlockSpec((T, D), lambda f: (0, 0)),
        out_shape=jax.ShapeDtypeStruct((T, D), jnp.float32),
    )(xf, shared_Wg, shared_Wu, shared_Wd)


def _sc_combine(shared_out, y, p0, p1):
    mesh = plsc.VectorSubcoreMesh(core_axis_name="c", subcore_axis_name="s")

    @functools.partial(
        pl.kernel,
        mesh=mesh,
        out_type=jax.ShapeDtypeStruct((T, D), jnp.float32),
        scratch_types=[
            pltpu.VMEM((S, D), jnp.float32),
            pltpu.VMEM((S, D), jnp.float32),
            pltpu.VMEM((S, D), jnp.float32),
            pltpu.VMEM((S,), jnp.int32),
            pltpu.VMEM((S,), jnp.int32),
            pltpu.SemaphoreType.DMA,
        ],
    )
    def k(sh_hbm, y_hbm, p0_hbm, p1_hbm, out_hbm,
          acc_v, y0_v, y1_v, i0_v, i1_v, sem):
        wid = lax.axis_index("s") * NC + lax.axis_index("c")
        base = wid * TPW
        for half in range(TPW // S):
            boff = base + half * S
            pltpu.sync_copy(sh_hbm.at[pl.ds(boff, S)], acc_v)
            pltpu.sync_copy(p0_hbm.at[pl.ds(boff, S)], i0_v)
            pltpu.sync_copy(p1_hbm.at[pl.ds(boff, S)], i1_v)
            c0 = pltpu.async_copy(y_hbm.at[i0_v], y0_v, sem)
            c1 = pltpu.async_copy(y_hbm.at[i1_v], y1_v, sem)
            c0.wait()
            c1.wait()

            def row_body(j, carry):
                for kk in range(D // 16):
                    sl = pl.ds(kk * 16, 16)
                    acc_v[j, sl] = acc_v[j, sl] + y0_v[j, sl] + y1_v[j, sl]
                return carry

            lax.fori_loop(0, S, row_body, 0)
            pltpu.sync_copy(acc_v, out_hbm.at[pl.ds(boff, S)])

    return k(shared_out, y, p0, p1)


def kernel(x, W_router, expert_bias, shared_Wg, shared_Wu, shared_Wd, Wg, Wu, Wd):
    b, s, d = x.shape
    xf = x.reshape(-1, d)

    w1, w2, p0, p1, te = _run_router(xf, W_router, expert_bias)
    p0f = p0.reshape(T)
    p1f = p1.reshape(T)
    w1f = w1.reshape(T)
    w2f = w2.reshape(T)
    tef = te.reshape(128)

    xs, rw = _sc_scatter(xf, p0f, p1f, w1f, w2f)
    y = _run_ffn(tef, xs, rw, Wg, Wu, Wd)
    shared_out = _run_shared(xf, shared_Wg, shared_Wu, shared_Wd)
    out = _sc_combine(shared_out, y, p0f, p1f)
    return out.reshape(b, s, d)


# speedup vs baseline: 1.4146x; 1.4146x over previous
import functools

import jax
import jax.numpy as jnp
from jax import lax
from jax.experimental import pallas as pl
from jax.experimental.pallas import tpu as pltpu
from jax.experimental.pallas import tpu_sc as plsc

D = 768
F = 2048
E = 8
T = 2048
TM = 256
NT = 24
P = NT * TM
FB = 512
NC = 2
NS = 16
NW = NC * NS
TPW = T // NW
S = 32
NEG = -1e30


def _router_body(wr_ref, x_ref, bias_ref, w1_ref, w2_ref, p0_ref, p1_ref,
                 te_ref):
    lT = lax.dot_general(wr_ref[...], x_ref[...], (((1,), (1,)), ((), ())),
                         preferred_element_type=jnp.float32)
    lT = lT + bias_ref[...]
    er = lax.broadcasted_iota(jnp.int32, lT.shape, 0)
    m1 = jnp.max(lT, axis=0, keepdims=True)
    a1 = jnp.min(jnp.where(lT == m1, er, E), axis=0, keepdims=True)
    oh1 = er == a1
    l2 = jnp.where(oh1, NEG, lT)
    m2 = jnp.max(l2, axis=0, keepdims=True)
    a2 = jnp.min(jnp.where(l2 == m2, er, E), axis=0, keepdims=True)
    oh2 = er == a2
    e21 = jnp.exp(m2 - m1)
    w1 = 1.0 / (1.0 + e21)
    w1_ref[...] = w1
    w2_ref[...] = 1.0 - w1

    C = jnp.where(jnp.logical_or(oh1, oh2), 1.0, 0.0)
    lane = lax.broadcasted_iota(jnp.int32, (E, T), 1)
    Cex = jnp.where(lane >= 1, pltpu.roll(C, 1, 1), 0.0)
    sh = 1
    while sh < T:
        Cex = Cex + jnp.where(lane >= sh + 1, pltpu.roll(Cex, sh, 1), 0.0)
        sh *= 2
    tot = jnp.max(Cex + C, axis=1, keepdims=True)
    pc = jnp.ceil(tot * (1.0 / TM)) * TM
    e8r = lax.broadcasted_iota(jnp.int32, (E, E), 0)
    e8c = lax.broadcasted_iota(jnp.int32, (E, E), 1)
    L8 = jnp.where(e8c < e8r, 1.0, 0.0)
    start = lax.dot_general(L8, pc, (((1,), (0,)), ((), ())),
                            preferred_element_type=jnp.float32)
    posbase = start + Cex
    p0 = jnp.sum(jnp.where(oh1, posbase, 0.0), axis=0, keepdims=True)
    p1 = jnp.sum(jnp.where(oh2, posbase, 0.0), axis=0, keepdims=True)
    p0_ref[...] = p0.astype(jnp.int32)
    p1_ref[...] = p1.astype(jnp.int32)

    ti = lax.broadcasted_iota(jnp.int32, (E, 128), 1).astype(jnp.float32) * TM
    inm = jnp.logical_and(ti >= start, ti < start + pc)
    ev = lax.broadcasted_iota(jnp.int32, (E, 128), 0).astype(jnp.float32)
    te = 7.0 + jnp.sum(jnp.where(inm, ev - 7.0, 0.0), axis=0, keepdims=True)
    ptot = jnp.max(start + pc)
    valid = jnp.where(ti[:1] < ptot, 1.0, 0.0)
    te_ref[...] = jnp.concatenate([te, valid], axis=0).astype(jnp.int32)


def _run_router(xf, W_router, expert_bias):
    return pl.pallas_call(
        _router_body,
        out_shape=[
            jax.ShapeDtypeStruct((1, T), jnp.float32),
            jax.ShapeDtypeStruct((1, T), jnp.float32),
            jax.ShapeDtypeStruct((1, T), jnp.int32),
            jax.ShapeDtypeStruct((1, T), jnp.int32),
            jax.ShapeDtypeStruct((2, 128), jnp.int32),
        ],
    )(W_router, xf, expert_bias.reshape(E, 1))


def _sc_scatter(xf, p0, p1, w0, w1):
    mesh = plsc.VectorSubcoreMesh(core_axis_name="c", subcore_axis_name="s")

    @functools.partial(
        pl.kernel,
        mesh=mesh,
        out_type=[
            jax.ShapeDtypeStruct((P, D), jnp.float32),
            jax.ShapeDtypeStruct((P,), jnp.float32),
        ],
        scratch_types=[
            pltpu.VMEM((TPW, D), jnp.float32),
            pltpu.VMEM((TPW,), jnp.int32),
            pltpu.VMEM((TPW,), jnp.int32),
            pltpu.VMEM((TPW,), jnp.float32),
            pltpu.VMEM((TPW,), jnp.float32),
            pltpu.SemaphoreType.DMA,
        ],
    )
    def k(x_hbm, p0_hbm, p1_hbm, w0_hbm, w1_hbm, xs_hbm, rw_hbm,
          rows_v, i0_v, i1_v, w0_v, w1_v, sem):
        wid = lax.axis_index("s") * NC + lax.axis_index("c")
        base = wid * TPW
        pltpu.sync_copy(x_hbm.at[pl.ds(base, TPW)], rows_v)
        pltpu.sync_copy(p0_hbm.at[pl.ds(base, TPW)], i0_v)
        pltpu.sync_copy(p1_hbm.at[pl.ds(base, TPW)], i1_v)
        pltpu.sync_copy(w0_hbm.at[pl.ds(base, TPW)], w0_v)
        pltpu.sync_copy(w1_hbm.at[pl.ds(base, TPW)], w1_v)
        outs = [
            pltpu.async_copy(rows_v, xs_hbm.at[i0_v], sem),
            pltpu.async_copy(rows_v, xs_hbm.at[i1_v], sem),
            pltpu.async_copy(w0_v, rw_hbm.at[i0_v], sem),
            pltpu.async_copy(w1_v, rw_hbm.at[i1_v], sem),
        ]
        for c in outs:
            c.wait()

    return k(xf, p0, p1, w0, w1)


def _ffn_body(te_ref, xs_ref, rw_ref, wg_ref, wu_ref, wd_ref, y_ref):
    i = pl.program_id(0)

    @pl.when(te_ref[128 + i] == 1)
    def _():
        xt = xs_ref[...].astype(jnp.bfloat16)
        g = lax.dot_general(xt, wg_ref[0].astype(jnp.bfloat16),
                            (((1,), (1,)), ((), ())),
                            preferred_element_type=jnp.float32)
        u = lax.dot_general(xt, wu_ref[0].astype(jnp.bfloat16),
                            (((1,), (1,)), ((), ())),
                            preferred_element_type=jnp.float32)
        h = (g * jax.nn.sigmoid(g) * u).astype(jnp.bfloat16)
        y = lax.dot_general(h, wd_ref[0].astype(jnp.bfloat16),
                            (((1,), (1,)), ((), ())),
                            preferred_element_type=jnp.float32)
        y_ref[...] = y * rw_ref[0]


def _run_ffn(te, xs, rw, Wg, Wu, Wd):
    return pl.pallas_call(
        _ffn_body,
        grid_spec=pltpu.PrefetchScalarGridSpec(
            num_scalar_prefetch=1,
            grid=(NT,),
            in_specs=[
                pl.BlockSpec((TM, D), lambda i, te: (i, 0)),
                pl.BlockSpec((1, TM, 1), lambda i, te: (i, 0, 0)),
                pl.BlockSpec((1, F, D), lambda i, te: (te[i], 0, 0)),
                pl.BlockSpec((1, F, D), lambda i, te: (te[i], 0, 0)),
                pl.BlockSpec((1, D, F), lambda i, te: (te[i], 0, 0)),
            ],
            out_specs=pl.BlockSpec((TM, D), lambda i, te: (i, 0)),
        ),
        out_shape=jax.ShapeDtypeStruct((P, D), jnp.float32),
    )(te, xs, rw.reshape(NT, TM, 1), Wg, Wu, Wd)


def _shared_body(x_ref, wg_ref, wu_ref, wd_ref, out_ref):
    f = pl.program_id(0)
    xt = x_ref[...].astype(jnp.bfloat16)
    g = lax.dot_general(xt, wg_ref[0].astype(jnp.bfloat16),
                        (((1,), (1,)), ((), ())),
                        preferred_element_type=jnp.float32)
    u = lax.dot_general(xt, wu_ref[0].astype(jnp.bfloat16),
                        (((1,), (1,)), ((), ())),
                        preferred_element_type=jnp.float32)
    h = (g * jax.nn.sigmoid(g) * u).astype(jnp.bfloat16)
    contrib = lax.dot_general(h, wd_ref[0].astype(jnp.bfloat16),
                              (((1,), (1,)), ((), ())),
                              preferred_element_type=jnp.float32)

    @pl.when(f == 0)
    def _():
        out_ref[...] = jnp.zeros_like(out_ref)

    out_ref[...] += contrib


def _run_shared(xf, shared_Wg, shared_Wu, shared_Wd):
    nf = F // FB
    return pl.pallas_call(
        _shared_body,
        grid=(nf,),
        in_specs=[
            pl.BlockSpec((T, D), lambda f: (0, 0)),
            pl.BlockSpec((1, FB, D), lambda f: (0, f, 0)),
            pl.BlockSpec((1, FB, D), lambda f: (0, f, 0)),
            pl.BlockSpec((1, D, FB), lambda f: (0, 0, f)),
        ],
        out_specs=pl.BlockSpec((T, D), lambda f: (0, 0)),
        out_shape=jax.ShapeDtypeStruct((T, D), jnp.float32),
    )(xf, shared_Wg, shared_Wu, shared_Wd)


def _sc_combine(shared_out, y, p0, p1):
    mesh = plsc.VectorSubcoreMesh(core_axis_name="c", subcore_axis_name="s")

    @functools.partial(
        pl.kernel,
        mesh=mesh,
        out_type=jax.ShapeDtypeStruct((T, D), jnp.float32),
        scratch_types=[
            pltpu.VMEM((S, D), jnp.float32),
            pltpu.VMEM((S, D), jnp.float32),
            pltpu.VMEM((S, D), jnp.float32),
            pltpu.VMEM((S,), jnp.int32),
            pltpu.VMEM((S,), jnp.int32),
            pltpu.SemaphoreType.DMA,
        ],
    )
    def k(sh_hbm, y_hbm, p0_hbm, p1_hbm, out_hbm,
          acc_v, y0_v, y1_v, i0_v, i1_v, sem):
        wid = lax.axis_index("s") * NC + lax.axis_index("c")
        base = wid * TPW
        for half in range(TPW // S):
            boff = base + half * S
            pltpu.sync_copy(sh_hbm.at[pl.ds(boff, S)], acc_v)
            pltpu.sync_copy(p0_hbm.at[pl.ds(boff, S)], i0_v)
            pltpu.sync_copy(p1_hbm.at[pl.ds(boff, S)], i1_v)
            c0 = pltpu.async_copy(y_hbm.at[i0_v], y0_v, sem)
            c1 = pltpu.async_copy(y_hbm.at[i1_v], y1_v, sem)
            c0.wait()
            c1.wait()

            def row_body(j, carry):
                for kk in range(D // 16):
                    sl = pl.ds(kk * 16, 16)
                    acc_v[j, sl] = acc_v[j, sl] + y0_v[j, sl] + y1_v[j, sl]
                return carry

            lax.fori_loop(0, S, row_body, 0)
            pltpu.sync_copy(acc_v, out_hbm.at[pl.ds(boff, S)])

    return k(shared_out, y, p0, p1)


def kernel(x, W_router, expert_bias, shared_Wg, shared_Wu, shared_Wd, Wg, Wu, Wd):
    b, s, d = x.shape
    xf = x.reshape(-1, d)

    w1, w2, p0, p1, te = _run_router(xf, W_router, expert_bias)
    p0f = p0.reshape(T)
    p1f = p1.reshape(T)
    w1f = w1.reshape(T)
    w2f = w2.reshape(T)
    tef = te.reshape(256)

    xs, rw = _sc_scatter(xf, p0f, p1f, w1f, w2f)
    y = _run_ffn(tef, xs, rw, Wg, Wu, Wd)
    shared_out = _run_shared(xf, shared_Wg, shared_Wu, shared_Wd)
    out = _sc_combine(shared_out, y, p0f, p1f)
    return out.reshape(b, s, d)


# speedup vs baseline: 1.4206x; 1.0043x over previous
import functools

import jax
import jax.numpy as jnp
from jax import lax
from jax.experimental import pallas as pl
from jax.experimental.pallas import tpu as pltpu
from jax.experimental.pallas import tpu_sc as plsc

D = 768
F = 2048
E = 8
T = 2048
TM = 256
NT = 24
P = NT * TM
FB = 512
NC = 2
NS = 16
NW = NC * NS
TPW = T // NW
S = 32
NEG = -1e30


def _router_body(wr_ref, x_ref, bias_ref, w1_ref, w2_ref, p0_ref, p1_ref,
                 te_ref):
    lT = lax.dot_general(wr_ref[...], x_ref[...], (((1,), (1,)), ((), ())),
                         preferred_element_type=jnp.float32)
    lT = lT + bias_ref[...]
    er = lax.broadcasted_iota(jnp.int32, lT.shape, 0)
    m1 = jnp.max(lT, axis=0, keepdims=True)
    a1 = jnp.min(jnp.where(lT == m1, er, E), axis=0, keepdims=True)
    oh1 = er == a1
    l2 = jnp.where(oh1, NEG, lT)
    m2 = jnp.max(l2, axis=0, keepdims=True)
    a2 = jnp.min(jnp.where(l2 == m2, er, E), axis=0, keepdims=True)
    oh2 = er == a2
    e21 = jnp.exp(m2 - m1)
    w1 = 1.0 / (1.0 + e21)
    w1_ref[...] = w1
    w2_ref[...] = 1.0 - w1

    C = jnp.where(jnp.logical_or(oh1, oh2), 1.0, 0.0)
    lane = lax.broadcasted_iota(jnp.int32, (E, T), 1)
    Cex = jnp.where(lane >= 1, pltpu.roll(C, 1, 1), 0.0)
    sh = 1
    while sh < T:
        Cex = Cex + jnp.where(lane >= sh + 1, pltpu.roll(Cex, sh, 1), 0.0)
        sh *= 2
    tot = jnp.max(Cex + C, axis=1, keepdims=True)
    pc = jnp.ceil(tot * (1.0 / TM)) * TM
    e8r = lax.broadcasted_iota(jnp.int32, (E, E), 0)
    e8c = lax.broadcasted_iota(jnp.int32, (E, E), 1)
    L8 = jnp.where(e8c < e8r, 1.0, 0.0)
    start = lax.dot_general(L8, pc, (((1,), (0,)), ((), ())),
                            preferred_element_type=jnp.float32)
    posbase = start + Cex
    p0 = jnp.sum(jnp.where(oh1, posbase, 0.0), axis=0, keepdims=True)
    p1 = jnp.sum(jnp.where(oh2, posbase, 0.0), axis=0, keepdims=True)
    p0_ref[...] = p0.astype(jnp.int32)
    p1_ref[...] = p1.astype(jnp.int32)

    ti = lax.broadcasted_iota(jnp.int32, (E, 128), 1).astype(jnp.float32) * TM
    inm = jnp.logical_and(ti >= start, ti < start + pc)
    ev = lax.broadcasted_iota(jnp.int32, (E, 128), 0).astype(jnp.float32)
    te = 7.0 + jnp.sum(jnp.where(inm, ev - 7.0, 0.0), axis=0, keepdims=True)
    ptot = jnp.max(start + pc)
    valid = jnp.where(ti[:1] < ptot, 1.0, 0.0)
    te_ref[...] = jnp.concatenate([te, valid], axis=0).astype(jnp.int32)


def _run_router(xf, W_router, expert_bias):
    return pl.pallas_call(
        _router_body,
        out_shape=[
            jax.ShapeDtypeStruct((1, T), jnp.float32),
            jax.ShapeDtypeStruct((1, T), jnp.float32),
            jax.ShapeDtypeStruct((1, T), jnp.int32),
            jax.ShapeDtypeStruct((1, T), jnp.int32),
            jax.ShapeDtypeStruct((2, 128), jnp.int32),
        ],
    )(W_router, xf, expert_bias.reshape(E, 1))


def _sc_scatter(xf, p0, p1, w0, w1):
    mesh = plsc.VectorSubcoreMesh(core_axis_name="c", subcore_axis_name="s")

    @functools.partial(
        pl.kernel,
        mesh=mesh,
        out_type=[
            jax.ShapeDtypeStruct((P, D), jnp.float32),
            jax.ShapeDtypeStruct((P,), jnp.float32),
        ],
        scratch_types=[
            pltpu.VMEM((TPW, D), jnp.float32),
            pltpu.VMEM((TPW,), jnp.int32),
            pltpu.VMEM((TPW,), jnp.int32),
            pltpu.VMEM((TPW,), jnp.float32),
            pltpu.VMEM((TPW,), jnp.float32),
            pltpu.SemaphoreType.DMA,
            pltpu.SemaphoreType.DMA,
        ],
    )
    def k(x_hbm, p0_hbm, p1_hbm, w0_hbm, w1_hbm, xs_hbm, rw_hbm,
          rows_v, i0_v, i1_v, w0_v, w1_v, sem, sem_in):
        wid = lax.axis_index("s") * NC + lax.axis_index("c")
        base = wid * TPW
        ins = [
            pltpu.async_copy(x_hbm.at[pl.ds(base, TPW)], rows_v, sem_in),
            pltpu.async_copy(p0_hbm.at[pl.ds(base, TPW)], i0_v, sem_in),
            pltpu.async_copy(p1_hbm.at[pl.ds(base, TPW)], i1_v, sem_in),
            pltpu.async_copy(w0_hbm.at[pl.ds(base, TPW)], w0_v, sem_in),
            pltpu.async_copy(w1_hbm.at[pl.ds(base, TPW)], w1_v, sem_in),
        ]
        for c in ins:
            c.wait()
        outs = [
            pltpu.async_copy(rows_v, xs_hbm.at[i0_v], sem),
            pltpu.async_copy(rows_v, xs_hbm.at[i1_v], sem),
            pltpu.async_copy(w0_v, rw_hbm.at[i0_v], sem),
            pltpu.async_copy(w1_v, rw_hbm.at[i1_v], sem),
        ]
        for c in outs:
            c.wait()

    return k(xf, p0, p1, w0, w1)


def _ffn_body(te_ref, xs_ref, rw_ref, wg_ref, wu_ref, wd_ref, y_ref):
    i = pl.program_id(0)

    @pl.when(te_ref[128 + i] == 1)
    def _():
        xt = xs_ref[...].astype(jnp.bfloat16)
        g = lax.dot_general(xt, wg_ref[0].astype(jnp.bfloat16),
                            (((1,), (1,)), ((), ())),
                            preferred_element_type=jnp.float32)
        u = lax.dot_general(xt, wu_ref[0].astype(jnp.bfloat16),
                            (((1,), (1,)), ((), ())),
                            preferred_element_type=jnp.float32)
        h = (g * jax.nn.sigmoid(g) * u).astype(jnp.bfloat16)
        y = lax.dot_general(h, wd_ref[0].astype(jnp.bfloat16),
                            (((1,), (1,)), ((), ())),
                            preferred_element_type=jnp.float32)
        y_ref[...] = y * rw_ref[0]


def _run_ffn(te, xs, rw, Wg, Wu, Wd):
    return pl.pallas_call(
        _ffn_body,
        grid_spec=pltpu.PrefetchScalarGridSpec(
            num_scalar_prefetch=1,
            grid=(NT,),
            in_specs=[
                pl.BlockSpec((TM, D), lambda i, te: (i, 0)),
                pl.BlockSpec((1, TM, 1), lambda i, te: (i, 0, 0)),
                pl.BlockSpec((1, F, D), lambda i, te: (te[i], 0, 0)),
                pl.BlockSpec((1, F, D), lambda i, te: (te[i], 0, 0)),
                pl.BlockSpec((1, D, F), lambda i, te: (te[i], 0, 0)),
            ],
            out_specs=pl.BlockSpec((TM, D), lambda i, te: (i, 0)),
        ),
        out_shape=jax.ShapeDtypeStruct((P, D), jnp.float32),
    )(te, xs, rw.reshape(NT, TM, 1), Wg, Wu, Wd)


def _shared_body(x_ref, wg_ref, wu_ref, wd_ref, out_ref):
    f = pl.program_id(0)
    xt = x_ref[...].astype(jnp.bfloat16)
    g = lax.dot_general(xt, wg_ref[0].astype(jnp.bfloat16),
                        (((1,), (1,)), ((), ())),
                        preferred_element_type=jnp.float32)
    u = lax.dot_general(xt, wu_ref[0].astype(jnp.bfloat16),
                        (((1,), (1,)), ((), ())),
                        preferred_element_type=jnp.float32)
    h = (g * jax.nn.sigmoid(g) * u).astype(jnp.bfloat16)
    contrib = lax.dot_general(h, wd_ref[0].astype(jnp.bfloat16),
                              (((1,), (1,)), ((), ())),
                              preferred_element_type=jnp.float32)

    @pl.when(f == 0)
    def _():
        out_ref[...] = jnp.zeros_like(out_ref)

    out_ref[...] += contrib


def _run_shared(xf, shared_Wg, shared_Wu, shared_Wd):
    nf = F // FB
    return pl.pallas_call(
        _shared_body,
        grid=(nf,),
        in_specs=[
            pl.BlockSpec((T, D), lambda f: (0, 0)),
            pl.BlockSpec((1, FB, D), lambda f: (0, f, 0)),
            pl.BlockSpec((1, FB, D), lambda f: (0, f, 0)),
            pl.BlockSpec((1, D, FB), lambda f: (0, 0, f)),
        ],
        out_specs=pl.BlockSpec((T, D), lambda f: (0, 0)),
        out_shape=jax.ShapeDtypeStruct((T, D), jnp.float32),
    )(xf, shared_Wg, shared_Wu, shared_Wd)


def _sc_combine(shared_out, y, p0, p1):
    mesh = plsc.VectorSubcoreMesh(core_axis_name="c", subcore_axis_name="s")

    @functools.partial(
        pl.kernel,
        mesh=mesh,
        out_type=jax.ShapeDtypeStruct((T, D), jnp.float32),
        scratch_types=[
            pltpu.VMEM((S, D), jnp.float32),
            pltpu.VMEM((S, D), jnp.float32),
            pltpu.VMEM((S, D), jnp.float32),
            pltpu.VMEM((S,), jnp.int32),
            pltpu.VMEM((S,), jnp.int32),
            pltpu.SemaphoreType.DMA,
        ],
    )
    def k(sh_hbm, y_hbm, p0_hbm, p1_hbm, out_hbm,
          acc_v, y0_v, y1_v, i0_v, i1_v, sem):
        wid = lax.axis_index("s") * NC + lax.axis_index("c")
        base = wid * TPW
        for half in range(TPW // S):
            boff = base + half * S
            pltpu.sync_copy(sh_hbm.at[pl.ds(boff, S)], acc_v)
            pltpu.sync_copy(p0_hbm.at[pl.ds(boff, S)], i0_v)
            pltpu.sync_copy(p1_hbm.at[pl.ds(boff, S)], i1_v)
            c0 = pltpu.async_copy(y_hbm.at[i0_v], y0_v, sem)
            c1 = pltpu.async_copy(y_hbm.at[i1_v], y1_v, sem)
            c0.wait()
            c1.wait()

            def row_body(j, carry):
                for kk in range(D // 16):
                    sl = pl.ds(kk * 16, 16)
                    acc_v[j, sl] = acc_v[j, sl] + y0_v[j, sl] + y1_v[j, sl]
                return carry

            lax.fori_loop(0, S, row_body, 0)
            pltpu.sync_copy(acc_v, out_hbm.at[pl.ds(boff, S)])

    return k(shared_out, y, p0, p1)


def kernel(x, W_router, expert_bias, shared_Wg, shared_Wu, shared_Wd, Wg, Wu, Wd):
    b, s, d = x.shape
    xf = x.reshape(-1, d)

    w1, w2, p0, p1, te = _run_router(xf, W_router, expert_bias)
    p0f = p0.reshape(T)
    p1f = p1.reshape(T)
    w1f = w1.reshape(T)
    w2f = w2.reshape(T)
    tef = te.reshape(256)

    shared_out = _run_shared(xf, shared_Wg, shared_Wu, shared_Wd)
    xs, rw = _sc_scatter(xf, p0f, p1f, w1f, w2f)
    y = _run_ffn(tef, xs, rw, Wg, Wu, Wd)
    out = _sc_combine(shared_out, y, p0f, p1f)
    return out.reshape(b, s, d)
